# baseline (speedup 1.0000x reference)
import functools

import jax
import jax.numpy as jnp
from jax import lax
from jax.experimental import pallas as pl
from jax.experimental.pallas import tpu as pltpu
from jax.experimental.pallas import tpu_sc as plsc

N = 10000
E = 320000
D = 128
DP = 144
NEG_SLOPE = 0.2

NC = 2
NS = 16
NW = NC * NS
C = 80
NCHUNK = 128
EPT = C * NCHUNK
E_PAD = NW * EPT
NA = N + 16
RPT = N // NS


def _dense_block(h, a2_ref, hp_ref, asrc_ref, adst_ref, bmax_ref):
    hp_ref[:, :D] = h
    col = lax.broadcasted_iota(jnp.int32, (N, DP - D), 1)
    hp_ref[:, D:] = jnp.where(col == 0, 1.0, 0.0)
    asd = jnp.dot(h, a2_ref[...], preferred_element_type=jnp.float32)
    asrc_ref[...] = asd[:, 0:1]
    adst_ref[...] = asd[:, 1:2]
    s = jnp.max(asd[:, 0:1]) + jnp.max(asd[:, 1:2])
    b = jnp.where(s > 0.0, s, NEG_SLOPE * s)
    bmax_ref[...] = jnp.full((16,), b, jnp.float32)


def _dense1_body(x_ref, w_ref, a2_ref, hp_ref, asrc_ref, adst_ref, bmax_ref):
    h = jnp.dot(x_ref[...], w_ref[...], preferred_element_type=jnp.float32)
    _dense_block(h, a2_ref, hp_ref, asrc_ref, adst_ref, bmax_ref)


def _mid_body(acc_ref, b1_ref, w_ref, a2_ref,
              hp_ref, asrc_ref, adst_ref, bmax_ref):
    acc = acc_ref[:N] + acc_ref[N:]
    h1 = acc[:, :D] / (acc[:, D:D + 1] + 1e-16) + b1_ref[...]
    h1 = jnp.where(h1 > 0.0, h1, jnp.exp(h1) - 1.0)
    h = jnp.dot(h1, w_ref[...], preferred_element_type=jnp.float32)
    _dense_block(h, a2_ref, hp_ref, asrc_ref, adst_ref, bmax_ref)


def _final_body(acc_ref, b2_ref, out_ref):
    acc = acc_ref[:N] + acc_ref[N:]
    out_ref[...] = acc[:, :D] / (acc[:, D:D + 1] + 1e-16) + b2_ref[...]


_stage_out = [
    jax.ShapeDtypeStruct((N, DP), jnp.float32),
    jax.ShapeDtypeStruct((N, 1), jnp.float32),
    jax.ShapeDtypeStruct((N, 1), jnp.float32),
    jax.ShapeDtypeStruct((16,), jnp.float32),
]

_dense1 = pl.pallas_call(_dense1_body, out_shape=_stage_out)
_mid = pl.pallas_call(_mid_body, out_shape=_stage_out)
_final = pl.pallas_call(
    _final_body, out_shape=jax.ShapeDtypeStruct((N, D), jnp.float32))


_mesh = plsc.VectorSubcoreMesh(core_axis_name="c", subcore_axis_name="s")


@functools.partial(
    pl.kernel,
    out_type=jax.ShapeDtypeStruct((NC * N, DP), jnp.float32),
    mesh=_mesh,
    scratch_types=[
        pltpu.VMEM((16,), jnp.float32),
        [pltpu.VMEM((C,), jnp.int32)] * 2,
        [pltpu.VMEM((C,), jnp.int32)] * 2,
        [pltpu.VMEM((C,), jnp.float32)] * 2,
        [pltpu.VMEM((C,), jnp.float32)] * 2,
        [pltpu.VMEM((C,), jnp.float32)] * 2,
        [pltpu.VMEM((C, DP), jnp.float32)] * 2,
        pltpu.VMEM_SHARED((NA, DP), jnp.float32),
        pltpu.SemaphoreType.DMA,
    ],
    compiler_params=pltpu.CompilerParams(
        use_tc_tiling_on_sc=False, needs_layout_passes=False),
)
def _edge_kernel(src_hbm, dst_hbm, asrc_hbm, adst_hbm, bmax_hbm, hp_hbm,
                 out_hbm, bmax_v, src_vs, dst_vs, asg_vs, adg_vs, w_vs,
                 rows_vs, acc_sp, sem):
    cid = lax.axis_index("c")
    sid = lax.axis_index("s")
    wid = sid * NC + cid

    pltpu.sync_copy(bmax_hbm, bmax_v)
    bmax = bmax_v[...]
    rows_v = rows_vs[0]

    zero16 = jnp.zeros((16,), jnp.float32)

    def _zbody(r, carry):
        for k in range(DP // 16):
            rows_v[r, pl.ds(k * 16, 16)] = zero16
        return carry

    lax.fori_loop(0, C, _zbody, 0)
    row0 = sid * RPT
    for i in range(RPT // C):
        pltpu.sync_copy(rows_v, acc_sp.at[pl.ds(row0 + i * C, C)])
    rem = RPT % C
    if rem:
        pltpu.sync_copy(rows_v.at[pl.ds(0, rem)],
                        acc_sp.at[pl.ds(row0 + RPT - rem, rem)])
    plsc.subcore_barrier()

    e0 = wid * EPT

    def _issue(ci, b):
        base = e0 + ci * C
        pltpu.sync_copy(src_hbm.at[pl.ds(base, C)], src_vs[b])
        pltpu.sync_copy(dst_hbm.at[pl.ds(base, C)], dst_vs[b])
        pltpu.async_copy(hp_hbm.at[src_vs[b]], rows_vs[b], sem)
        pltpu.async_copy(asrc_hbm.at[src_vs[b]], asg_vs[b], sem)
        pltpu.async_copy(adst_hbm.at[dst_vs[b]], adg_vs[b], sem)

    def _consume(b):
        pltpu.make_async_copy(hp_hbm.at[src_vs[b]], rows_vs[b], sem).wait()
        pltpu.make_async_copy(asrc_hbm.at[src_vs[b]], asg_vs[b], sem).wait()
        pltpu.make_async_copy(adst_hbm.at[dst_vs[b]], adg_vs[b], sem).wait()
        for j in range(C // 16):
            e = (asg_vs[b][pl.ds(j * 16, 16)]
                 + adg_vs[b][pl.ds(j * 16, 16)])
            e = jnp.where(e > 0.0, e, NEG_SLOPE * e)
            w_vs[b][pl.ds(j * 16, 16)] = jnp.exp(e - bmax)

        def _scale(r, c2):
            wb = plsc.load_gather(w_vs[b], [jnp.full((16,), r, jnp.int32)])
            for k in range(DP // 16):
                rows_vs[b][r, pl.ds(k * 16, 16)] = (
                    rows_vs[b][r, pl.ds(k * 16, 16)] * wb)
            return c2

        lax.fori_loop(0, C, _scale, 0)
        pltpu.sync_copy(rows_vs[b], acc_sp.at[dst_vs[b]], add=True)

    _issue(0, 0)

    def _pair(k, carry):
        ci = 2 * k
        _issue(ci + 1, 1)
        _consume(0)
        _issue(ci + 2, 0)
        _consume(1)
        return carry

    lax.fori_loop(0, NCHUNK // 2 - 1, _pair, 0)
    _issue(NCHUNK - 1, 1)
    _consume(0)
    _consume(1)
    plsc.subcore_barrier()

    out0 = cid * N + row0
    for i in range(RPT // C):
        pltpu.sync_copy(acc_sp.at[pl.ds(row0 + i * C, C)], rows_v)
        pltpu.sync_copy(rows_v, out_hbm.at[pl.ds(out0 + i * C, C)])
    if rem:
        pltpu.sync_copy(acc_sp.at[pl.ds(row0 + RPT - rem, rem)],
                        rows_v.at[pl.ds(0, rem)])
        pltpu.sync_copy(rows_v.at[pl.ds(0, rem)],
                        out_hbm.at[pl.ds(out0 + RPT - rem, rem)])


def kernel(x, edge_index, W1, a_src1, a_dst1, b1, W2, a_src2, a_dst2, b2):
    pad = E_PAD - E
    src = jnp.concatenate([edge_index[0], jnp.zeros((pad,), jnp.int32)])
    dst = jnp.concatenate(
        [edge_index[1], N + (jnp.arange(pad, dtype=jnp.int32) % 16)])
    a21 = jnp.concatenate([a_src1, a_dst1], axis=0).T
    a22 = jnp.concatenate([a_src2, a_dst2], axis=0).T

    hp1, asrc1, adst1, bmax1 = _dense1(x, W1, a21)
    acc1 = _edge_kernel(src, dst, asrc1.reshape(N), adst1.reshape(N),
                        bmax1, hp1)
    hp2, asrc2, adst2, bmax2 = _mid(acc1, b1, W2, a22)
    acc2 = _edge_kernel(src, dst, asrc2.reshape(N), adst2.reshape(N),
                        bmax2, hp2)
    return _final(acc2, b2)


# speedup vs baseline: 1.7165x; 1.7165x over previous
import functools

import jax
import jax.numpy as jnp
from jax import lax
from jax.experimental import pallas as pl
from jax.experimental.pallas import tpu as pltpu
from jax.experimental.pallas import tpu_sc as plsc

N = 10000
E = 320000
D = 128
DP = 144
NEG_SLOPE = 0.2

NC = 2
NS = 16
NW = NC * NS
C = 80
NCHUNK = 128
EPT = C * NCHUNK
E_PAD = NW * EPT
NA = N + C
RPT = N // NS


def _dense_block(h, a2_ref, hp_ref, asrc_ref, adst_ref, bmax_ref):
    hp_ref[:, :D] = h
    col = lax.broadcasted_iota(jnp.int32, (N, DP - D), 1)
    hp_ref[:, D:] = jnp.where(col == 0, 1.0, 0.0)
    asd = jnp.dot(h, a2_ref[...], preferred_element_type=jnp.float32)
    asrc_ref[...] = asd[:, 0:1]
    adst_ref[...] = asd[:, 1:2]
    s = jnp.max(asd[:, 0:1]) + jnp.max(asd[:, 1:2])
    b = jnp.where(s > 0.0, s, NEG_SLOPE * s)
    bmax_ref[...] = jnp.full((16,), b, jnp.float32)


def _dense1_body(x_ref, w_ref, a2_ref, hp_ref, asrc_ref, adst_ref, bmax_ref):
    h = jnp.dot(x_ref[...], w_ref[...], preferred_element_type=jnp.float32)
    _dense_block(h, a2_ref, hp_ref, asrc_ref, adst_ref, bmax_ref)


def _mid_body(acc_ref, b1_ref, w_ref, a2_ref,
              hp_ref, asrc_ref, adst_ref, bmax_ref):
    acc = acc_ref[:N] + acc_ref[N:]
    h1 = acc[:, :D] / (acc[:, D:D + 1] + 1e-16) + b1_ref[...]
    h1 = jnp.where(h1 > 0.0, h1, jnp.exp(h1) - 1.0)
    h = jnp.dot(h1, w_ref[...], preferred_element_type=jnp.float32)
    _dense_block(h, a2_ref, hp_ref, asrc_ref, adst_ref, bmax_ref)


def _final_body(acc_ref, b2_ref, out_ref):
    acc = acc_ref[:N] + acc_ref[N:]
    out_ref[...] = acc[:, :D] / (acc[:, D:D + 1] + 1e-16) + b2_ref[...]


_stage_out = [
    jax.ShapeDtypeStruct((N, DP), jnp.float32),
    jax.ShapeDtypeStruct((N, 1), jnp.float32),
    jax.ShapeDtypeStruct((N, 1), jnp.float32),
    jax.ShapeDtypeStruct((16,), jnp.float32),
]

_dense1 = pl.pallas_call(_dense1_body, out_shape=_stage_out)
_mid = pl.pallas_call(_mid_body, out_shape=_stage_out)
_final = pl.pallas_call(
    _final_body, out_shape=jax.ShapeDtypeStruct((N, D), jnp.float32))


_mesh = plsc.VectorSubcoreMesh(core_axis_name="c", subcore_axis_name="s")


@functools.partial(
    pl.kernel,
    out_type=jax.ShapeDtypeStruct((NC * N, DP), jnp.float32),
    mesh=_mesh,
    scratch_types=[
        pltpu.VMEM((16,), jnp.float32),
        [pltpu.VMEM((C,), jnp.int32)] * 2,
        [pltpu.VMEM((C,), jnp.int32)] * 2,
        [pltpu.VMEM((C,), jnp.float32)] * 2,
        [pltpu.VMEM((C,), jnp.float32)] * 2,
        [pltpu.VMEM((C,), jnp.float32)] * 2,
        [pltpu.VMEM((C, DP), jnp.float32)] * 2,
        pltpu.VMEM_SHARED((NA, DP), jnp.float32),
        pltpu.SemaphoreType.DMA,
    ],
    compiler_params=pltpu.CompilerParams(
        use_tc_tiling_on_sc=False, needs_layout_passes=False),
)
def _edge_kernel(src_hbm, dst_hbm, asrc_hbm, adst_hbm, bmax_hbm, hp_hbm,
                 out_hbm, bmax_v, src_vs, dst_vs, asg_vs, adg_vs, w_vs,
                 rows_vs, acc_sp, sem):
    cid = lax.axis_index("c")
    sid = lax.axis_index("s")
    wid = sid * NC + cid

    pltpu.sync_copy(bmax_hbm, bmax_v)
    bmax = bmax_v[...]
    rows_v = rows_vs[0]

    zero16 = jnp.zeros((16,), jnp.float32)

    def _zbody(r, carry):
        for k in range(DP // 16):
            rows_v[r, pl.ds(k * 16, 16)] = zero16
        return carry

    lax.fori_loop(0, C, _zbody, 0)
    row0 = sid * RPT
    for i in range(RPT // C):
        pltpu.sync_copy(rows_v, acc_sp.at[pl.ds(row0 + i * C, C)])
    rem = RPT % C
    if rem:
        pltpu.sync_copy(rows_v.at[pl.ds(0, rem)],
                        acc_sp.at[pl.ds(row0 + RPT - rem, rem)])
    plsc.subcore_barrier()

    e0 = wid * EPT

    def _issue(ci, b):
        base = e0 + ci * C
        pltpu.sync_copy(src_hbm.at[pl.ds(base, C)], src_vs[b])
        pltpu.sync_copy(dst_hbm.at[pl.ds(base, C)], dst_vs[b])
        pltpu.async_copy(hp_hbm.at[src_vs[b]], rows_vs[b], sem)
        pltpu.async_copy(asrc_hbm.at[src_vs[b]], asg_vs[b], sem)
        pltpu.async_copy(adst_hbm.at[dst_vs[b]], adg_vs[b], sem)

    def _consume(b):
        pltpu.make_async_copy(hp_hbm.at[src_vs[b]], rows_vs[b], sem).wait()
        pltpu.make_async_copy(asrc_hbm.at[src_vs[b]], asg_vs[b], sem).wait()
        pltpu.make_async_copy(adst_hbm.at[dst_vs[b]], adg_vs[b], sem).wait()
        for j in range(C // 16):
            e = (asg_vs[b][pl.ds(j * 16, 16)]
                 + adg_vs[b][pl.ds(j * 16, 16)])
            e = jnp.where(e > 0.0, e, NEG_SLOPE * e)
            w_vs[b][pl.ds(j * 16, 16)] = jnp.exp(e - bmax)

        def _scale(r, c2):
            wb = plsc.load_gather(w_vs[b], [jnp.full((16,), r, jnp.int32)])
            for k in range(DP // 16):
                rows_vs[b][r, pl.ds(k * 16, 16)] = (
                    rows_vs[b][r, pl.ds(k * 16, 16)] * wb)
            return c2

        lax.fori_loop(0, C, _scale, 0)
        pltpu.sync_copy(rows_vs[b], acc_sp.at[dst_vs[b]], add=True)

    _issue(0, 0)

    def _pair(k, carry):
        ci = 2 * k
        _issue(ci + 1, 1)
        _consume(0)
        _issue(ci + 2, 0)
        _consume(1)
        return carry

    lax.fori_loop(0, NCHUNK // 2 - 1, _pair, 0)
    _issue(NCHUNK - 1, 1)
    _consume(0)
    _consume(1)
    plsc.subcore_barrier()

    out0 = cid * N + row0
    for i in range(RPT // C):
        pltpu.sync_copy(acc_sp.at[pl.ds(row0 + i * C, C)], rows_v)
        pltpu.sync_copy(rows_v, out_hbm.at[pl.ds(out0 + i * C, C)])
    if rem:
        pltpu.sync_copy(acc_sp.at[pl.ds(row0 + RPT - rem, rem)],
                        rows_v.at[pl.ds(0, rem)])
        pltpu.sync_copy(rows_v.at[pl.ds(0, rem)],
                        out_hbm.at[pl.ds(out0 + RPT - rem, rem)])


def kernel(x, edge_index, W1, a_src1, a_dst1, b1, W2, a_src2, a_dst2, b2):
    pad = E_PAD - E
    ar = jnp.arange(pad, dtype=jnp.int32)
    src = jnp.concatenate([edge_index[0], ar % N])
    dst = jnp.concatenate([edge_index[1], N + (ar % C)])
    a21 = jnp.concatenate([a_src1, a_dst1], axis=0).T
    a22 = jnp.concatenate([a_src2, a_dst2], axis=0).T

    zpad = jnp.zeros((NA - N,), jnp.float32)

    hp1, asrc1, adst1, bmax1 = _dense1(x, W1, a21)
    acc1 = _edge_kernel(src, dst, asrc1.reshape(N),
                        jnp.concatenate([adst1.reshape(N), zpad]),
                        bmax1, hp1)
    hp2, asrc2, adst2, bmax2 = _mid(acc1, b1, W2, a22)
    acc2 = _edge_kernel(src, dst, asrc2.reshape(N),
                        jnp.concatenate([adst2.reshape(N), zpad]),
                        bmax2, hp2)
    return _final(acc2, b2)


# speedup vs baseline: 1.9174x; 1.1170x over previous
import functools

import jax
import jax.numpy as jnp
from jax import lax
from jax.experimental import pallas as pl
from jax.experimental.pallas import tpu as pltpu
from jax.experimental.pallas import tpu_sc as plsc

N = 10000
E = 320000
D = 128
DP = 144
NEG_SLOPE = 0.2

NC = 2
NS = 16
NW = NC * NS
C = 128
NCHUNK = 80
EPT = C * NCHUNK
E_PAD = NW * EPT
NA = N + C
RPT = N // NS


def _dense_block(h, a2_ref, hp_ref, asrc_ref, adst_ref, bmax_ref):
    hp_ref[:, :D] = h
    col = lax.broadcasted_iota(jnp.int32, (N, DP - D), 1)
    hp_ref[:, D:] = jnp.where(col == 0, 1.0, 0.0)
    asd = jnp.dot(h, a2_ref[...], preferred_element_type=jnp.float32)
    asrc_ref[...] = asd[:, 0:1]
    adst_ref[...] = asd[:, 1:2]
    s = jnp.max(asd[:, 0:1]) + jnp.max(asd[:, 1:2])
    b = jnp.where(s > 0.0, s, NEG_SLOPE * s)
    bmax_ref[...] = jnp.full((16,), b, jnp.float32)


def _dense1_body(x_ref, w_ref, a2_ref, hp_ref, asrc_ref, adst_ref, bmax_ref):
    h = jnp.dot(x_ref[...], w_ref[...], preferred_element_type=jnp.float32)
    _dense_block(h, a2_ref, hp_ref, asrc_ref, adst_ref, bmax_ref)


def _mid_body(acc_ref, b1_ref, w_ref, a2_ref,
              hp_ref, asrc_ref, adst_ref, bmax_ref):
    acc = acc_ref[:N] + acc_ref[N:]
    h1 = acc[:, :D] / (acc[:, D:D + 1] + 1e-16) + b1_ref[...]
    h1 = jnp.where(h1 > 0.0, h1, jnp.exp(h1) - 1.0)
    h = jnp.dot(h1, w_ref[...], preferred_element_type=jnp.float32)
    _dense_block(h, a2_ref, hp_ref, asrc_ref, adst_ref, bmax_ref)


def _final_body(acc_ref, b2_ref, out_ref):
    acc = acc_ref[:N] + acc_ref[N:]
    out_ref[...] = acc[:, :D] / (acc[:, D:D + 1] + 1e-16) + b2_ref[...]


_stage_out = [
    jax.ShapeDtypeStruct((N, DP), jnp.float32),
    jax.ShapeDtypeStruct((N, 1), jnp.float32),
    jax.ShapeDtypeStruct((N, 1), jnp.float32),
    jax.ShapeDtypeStruct((16,), jnp.float32),
]

_dense1 = pl.pallas_call(_dense1_body, out_shape=_stage_out)
_mid = pl.pallas_call(_mid_body, out_shape=_stage_out)
_final = pl.pallas_call(
    _final_body, out_shape=jax.ShapeDtypeStruct((N, D), jnp.float32))


_mesh = plsc.VectorSubcoreMesh(core_axis_name="c", subcore_axis_name="s")


@functools.partial(
    pl.kernel,
    out_type=jax.ShapeDtypeStruct((NC * N, DP), jnp.float32),
    mesh=_mesh,
    scratch_types=[
        pltpu.VMEM((16,), jnp.float32),
        [pltpu.VMEM((C,), jnp.int32)] * 2,
        [pltpu.VMEM((C,), jnp.int32)] * 2,
        [pltpu.VMEM((C,), jnp.float32)] * 2,
        [pltpu.VMEM((C,), jnp.float32)] * 2,
        [pltpu.VMEM((C,), jnp.float32)] * 2,
        [pltpu.VMEM((C, DP), jnp.float32)] * 2,
        pltpu.VMEM_SHARED((NA, DP), jnp.float32),
        [pltpu.SemaphoreType.DMA] * 2,
        [pltpu.SemaphoreType.DMA] * 2,
    ],
    compiler_params=pltpu.CompilerParams(
        use_tc_tiling_on_sc=False, needs_layout_passes=False),
)
def _edge_kernel(src_hbm, dst_hbm, asrc_hbm, adst_hbm, bmax_hbm, hp_hbm,
                 out_hbm, bmax_v, src_vs, dst_vs, asg_vs, adg_vs, w_vs,
                 rows_vs, acc_sp, gsems, ssems):
    cid = lax.axis_index("c")
    sid = lax.axis_index("s")
    wid = sid * NC + cid

    pltpu.sync_copy(bmax_hbm, bmax_v)
    bmax = bmax_v[...]
    rows_v = rows_vs[0]

    zero16 = jnp.zeros((16,), jnp.float32)

    def _zbody(r, carry):
        for k in range(DP // 16):
            rows_v[r, pl.ds(k * 16, 16)] = zero16
        return carry

    lax.fori_loop(0, C, _zbody, 0)
    row0 = sid * RPT
    for i in range(RPT // C):
        pltpu.sync_copy(rows_v, acc_sp.at[pl.ds(row0 + i * C, C)])
    rem = RPT % C
    if rem:
        pltpu.sync_copy(rows_v.at[pl.ds(0, rem)],
                        acc_sp.at[pl.ds(row0 + RPT - rem, rem)])
    plsc.subcore_barrier()

    e0 = wid * EPT

    def _issue(ci, b, drain):
        if drain:
            pltpu.make_async_copy(rows_vs[b], acc_sp.at[dst_vs[b]],
                                  ssems[b]).wait()
        base = e0 + ci * C
        pltpu.sync_copy(src_hbm.at[pl.ds(base, C)], src_vs[b])
        pltpu.sync_copy(dst_hbm.at[pl.ds(base, C)], dst_vs[b])
        pltpu.async_copy(hp_hbm.at[src_vs[b]], rows_vs[b], gsems[b])
        pltpu.async_copy(asrc_hbm.at[src_vs[b]], asg_vs[b], gsems[b])
        pltpu.async_copy(adst_hbm.at[dst_vs[b]], adg_vs[b], gsems[b])

    def _consume(b):
        pltpu.make_async_copy(hp_hbm.at[src_vs[b]], rows_vs[b],
                              gsems[b]).wait()
        pltpu.make_async_copy(asrc_hbm.at[src_vs[b]], asg_vs[b],
                              gsems[b]).wait()
        pltpu.make_async_copy(adst_hbm.at[dst_vs[b]], adg_vs[b],
                              gsems[b]).wait()
        for j in range(C // 16):
            e = (asg_vs[b][pl.ds(j * 16, 16)]
                 + adg_vs[b][pl.ds(j * 16, 16)])
            e = jnp.where(e > 0.0, e, NEG_SLOPE * e)
            w_vs[b][pl.ds(j * 16, 16)] = jnp.exp(e - bmax)

        def _scale(r, c2):
            wb = plsc.load_gather(w_vs[b], [jnp.full((16,), r, jnp.int32)])
            for k in range(DP // 16):
                rows_vs[b][r, pl.ds(k * 16, 16)] = (
                    rows_vs[b][r, pl.ds(k * 16, 16)] * wb)
            return c2

        lax.fori_loop(0, C, _scale, 0)
        pltpu.async_copy(rows_vs[b], acc_sp.at[dst_vs[b]], ssems[b],
                         add=True)

    _issue(0, 0, False)
    _issue(1, 1, False)

    def _pair(k, carry):
        ci = 2 * k
        _consume(0)
        _issue(ci + 2, 0, True)
        _consume(1)
        _issue(ci + 3, 1, True)
        return carry

    lax.fori_loop(0, NCHUNK // 2 - 1, _pair, 0)
    _consume(0)
    _consume(1)
    pltpu.make_async_copy(rows_vs[0], acc_sp.at[dst_vs[0]], ssems[0]).wait()
    pltpu.make_async_copy(rows_vs[1], acc_sp.at[dst_vs[1]], ssems[1]).wait()
    plsc.subcore_barrier()

    out0 = cid * N + row0
    for i in range(RPT // C):
        pltpu.sync_copy(acc_sp.at[pl.ds(row0 + i * C, C)], rows_v)
        pltpu.sync_copy(rows_v, out_hbm.at[pl.ds(out0 + i * C, C)])
    if rem:
        pltpu.sync_copy(acc_sp.at[pl.ds(row0 + RPT - rem, rem)],
                        rows_v.at[pl.ds(0, rem)])
        pltpu.sync_copy(rows_v.at[pl.ds(0, rem)],
                        out_hbm.at[pl.ds(out0 + RPT - rem, rem)])


def kernel(x, edge_index, W1, a_src1, a_dst1, b1, W2, a_src2, a_dst2, b2):
    pad = E_PAD - E
    ar = jnp.arange(pad, dtype=jnp.int32)
    src = jnp.concatenate([edge_index[0], ar % N])
    dst = jnp.concatenate([edge_index[1], N + (ar % C)])
    a21 = jnp.concatenate([a_src1, a_dst1], axis=0).T
    a22 = jnp.concatenate([a_src2, a_dst2], axis=0).T

    zpad = jnp.zeros((NA - N,), jnp.float32)

    hp1, asrc1, adst1, bmax1 = _dense1(x, W1, a21)
    acc1 = _edge_kernel(src, dst, asrc1.reshape(N),
                        jnp.concatenate([adst1.reshape(N), zpad]),
                        bmax1, hp1)
    hp2, asrc2, adst2, bmax2 = _mid(acc1, b1, W2, a22)
    acc2 = _edge_kernel(src, dst, asrc2.reshape(N),
                        jnp.concatenate([adst2.reshape(N), zpad]),
                        bmax2, hp2)
    return _final(acc2, b2)


# speedup vs baseline: 2.2076x; 1.1514x over previous
import functools

import jax
import jax.numpy as jnp
from jax import lax
from jax.experimental import pallas as pl
from jax.experimental.pallas import tpu as pltpu
from jax.experimental.pallas import tpu_sc as plsc

N = 10000
E = 320000
D = 128
DP = 144
NEG_SLOPE = 0.2

NC = 2
NS = 16
NW = NC * NS
C = 128
NCHUNK = 80
EPT = C * NCHUNK
E_PAD = NW * EPT
NA = N + C
RPT = N // NS


def _dense_block(h, a2_ref, hp_ref, asrc_ref, adst_ref, bmax_ref):
    hp_ref[:, :D] = h
    col = lax.broadcasted_iota(jnp.int32, (N, DP - D), 1)
    hp_ref[:, D:] = jnp.where(col == 0, 1.0, 0.0)
    asd = jnp.dot(h, a2_ref[...], preferred_element_type=jnp.float32)
    asrc_ref[...] = asd[:, 0:1]
    adst_ref[...] = asd[:, 1:2]
    s = jnp.max(asd[:, 0:1]) + jnp.max(asd[:, 1:2])
    b = jnp.where(s > 0.0, s, NEG_SLOPE * s)
    bmax_ref[...] = jnp.full((16,), b, jnp.float32)


def _dense1_body(x_ref, w_ref, a2_ref, hp_ref, asrc_ref, adst_ref, bmax_ref):
    h = jnp.dot(x_ref[...], w_ref[...], preferred_element_type=jnp.float32)
    _dense_block(h, a2_ref, hp_ref, asrc_ref, adst_ref, bmax_ref)


def _mid_body(acc_ref, b1_ref, w_ref, a2_ref,
              hp_ref, asrc_ref, adst_ref, bmax_ref):
    acc = acc_ref[:N] + acc_ref[N:]
    h1 = acc[:, :D] / (acc[:, D:D + 1] + 1e-16) + b1_ref[...]
    h1 = jnp.where(h1 > 0.0, h1, jnp.exp(h1) - 1.0)
    h = jnp.dot(h1, w_ref[...], preferred_element_type=jnp.float32)
    _dense_block(h, a2_ref, hp_ref, asrc_ref, adst_ref, bmax_ref)


def _final_body(acc_ref, b2_ref, out_ref):
    acc = acc_ref[:N] + acc_ref[N:]
    out_ref[...] = acc[:, :D] / (acc[:, D:D + 1] + 1e-16) + b2_ref[...]


_stage_out = [
    jax.ShapeDtypeStruct((N, DP), jnp.float32),
    jax.ShapeDtypeStruct((N, 1), jnp.float32),
    jax.ShapeDtypeStruct((N, 1), jnp.float32),
    jax.ShapeDtypeStruct((16,), jnp.float32),
]

_dense1 = pl.pallas_call(_dense1_body, out_shape=_stage_out)
_mid = pl.pallas_call(_mid_body, out_shape=_stage_out)
_final = pl.pallas_call(
    _final_body, out_shape=jax.ShapeDtypeStruct((N, D), jnp.float32))


_mesh = plsc.VectorSubcoreMesh(core_axis_name="c", subcore_axis_name="s")


@functools.partial(
    pl.kernel,
    out_type=jax.ShapeDtypeStruct((NC * N, DP), jnp.float32),
    mesh=_mesh,
    scratch_types=[
        pltpu.VMEM((16,), jnp.float32),
        [pltpu.VMEM((C,), jnp.int32)] * 2,
        [pltpu.VMEM((C,), jnp.int32)] * 2,
        [pltpu.VMEM((C,), jnp.float32)] * 2,
        [pltpu.VMEM((C,), jnp.float32)] * 2,
        [pltpu.VMEM((C,), jnp.float32)] * 2,
        [pltpu.VMEM((C, DP), jnp.float32)] * 2,
        pltpu.VMEM_SHARED((NA, DP), jnp.float32),
        [pltpu.SemaphoreType.DMA] * 2,
        [pltpu.SemaphoreType.DMA] * 2,
    ],
    compiler_params=pltpu.CompilerParams(
        use_tc_tiling_on_sc=False, needs_layout_passes=False),
)
def _edge_kernel(src_hbm, dst_hbm, asrc_hbm, adst_hbm, bmax_hbm, hp_hbm,
                 out_hbm, bmax_v, src_vs, dst_vs, asg_vs, adg_vs, w_vs,
                 rows_vs, acc_sp, gsems, ssems):
    cid = lax.axis_index("c")
    sid = lax.axis_index("s")
    wid = sid * NC + cid

    pltpu.sync_copy(bmax_hbm, bmax_v)
    bmax = bmax_v[...]
    rows_v = rows_vs[0]

    zero16 = jnp.zeros((16,), jnp.float32)

    def _zbody(r, carry):
        for k in range(DP // 16):
            rows_v[r, pl.ds(k * 16, 16)] = zero16
        return carry

    lax.fori_loop(0, C, _zbody, 0)
    row0 = sid * RPT
    for i in range(RPT // C):
        pltpu.sync_copy(rows_v, acc_sp.at[pl.ds(row0 + i * C, C)])
    rem = RPT % C
    if rem:
        pltpu.sync_copy(rows_v.at[pl.ds(0, rem)],
                        acc_sp.at[pl.ds(row0 + RPT - rem, rem)])
    plsc.subcore_barrier()

    e0 = wid * EPT

    def _issue(ci, b, drain):
        if drain:
            pltpu.make_async_copy(rows_vs[b], acc_sp.at[dst_vs[b]],
                                  ssems[b]).wait()
        base = e0 + ci * C
        pltpu.sync_copy(src_hbm.at[pl.ds(base, C)], src_vs[b])
        pltpu.sync_copy(dst_hbm.at[pl.ds(base, C)], dst_vs[b])
        pltpu.async_copy(hp_hbm.at[src_vs[b]], rows_vs[b], gsems[b])
        pltpu.async_copy(asrc_hbm.at[src_vs[b]], asg_vs[b], gsems[b])
        pltpu.async_copy(adst_hbm.at[dst_vs[b]], adg_vs[b], gsems[b])

    def _consume(b):
        pltpu.make_async_copy(hp_hbm.at[src_vs[b]], rows_vs[b],
                              gsems[b]).wait()
        pltpu.make_async_copy(asrc_hbm.at[src_vs[b]], asg_vs[b],
                              gsems[b]).wait()
        pltpu.make_async_copy(adst_hbm.at[dst_vs[b]], adg_vs[b],
                              gsems[b]).wait()
        for j in range(C // 16):
            e = (asg_vs[b][pl.ds(j * 16, 16)]
                 + adg_vs[b][pl.ds(j * 16, 16)])
            e = jnp.where(e > 0.0, e, NEG_SLOPE * e)
            w_vs[b][pl.ds(j * 16, 16)] = jnp.exp(e - bmax)

        @plsc.parallel_loop(0, C, step=1, unroll=4)
        def _scale(r):
            wb = plsc.load_gather(w_vs[b], [jnp.full((16,), r, jnp.int32)])
            for k in range(DP // 16):
                rows_vs[b][r, pl.ds(k * 16, 16)] = (
                    rows_vs[b][r, pl.ds(k * 16, 16)] * wb)

        pltpu.async_copy(rows_vs[b], acc_sp.at[dst_vs[b]], ssems[b],
                         add=True)

    _issue(0, 0, False)
    _issue(1, 1, False)

    def _pair(k, carry):
        ci = 2 * k
        _consume(0)
        _issue(ci + 2, 0, True)
        _consume(1)
        _issue(ci + 3, 1, True)
        return carry

    lax.fori_loop(0, NCHUNK // 2 - 1, _pair, 0)
    _consume(0)
    _consume(1)
    pltpu.make_async_copy(rows_vs[0], acc_sp.at[dst_vs[0]], ssems[0]).wait()
    pltpu.make_async_copy(rows_vs[1], acc_sp.at[dst_vs[1]], ssems[1]).wait()
    plsc.subcore_barrier()

    out0 = cid * N + row0
    for i in range(RPT // C):
        pltpu.sync_copy(acc_sp.at[pl.ds(row0 + i * C, C)], rows_v)
        pltpu.sync_copy(rows_v, out_hbm.at[pl.ds(out0 + i * C, C)])
    if rem:
        pltpu.sync_copy(acc_sp.at[pl.ds(row0 + RPT - rem, rem)],
                        rows_v.at[pl.ds(0, rem)])
        pltpu.sync_copy(rows_v.at[pl.ds(0, rem)],
                        out_hbm.at[pl.ds(out0 + RPT - rem, rem)])


def kernel(x, edge_index, W1, a_src1, a_dst1, b1, W2, a_src2, a_dst2, b2):
    pad = E_PAD - E
    ar = jnp.arange(pad, dtype=jnp.int32)
    src = jnp.concatenate([edge_index[0], ar % N])
    dst = jnp.concatenate([edge_index[1], N + (ar % C)])
    a21 = jnp.concatenate([a_src1, a_dst1], axis=0).T
    a22 = jnp.concatenate([a_src2, a_dst2], axis=0).T

    zpad = jnp.zeros((NA - N,), jnp.float32)

    hp1, asrc1, adst1, bmax1 = _dense1(x, W1, a21)
    acc1 = _edge_kernel(src, dst, asrc1.reshape(N),
                        jnp.concatenate([adst1.reshape(N), zpad]),
                        bmax1, hp1)
    hp2, asrc2, adst2, bmax2 = _mid(acc1, b1, W2, a22)
    acc2 = _edge_kernel(src, dst, asrc2.reshape(N),
                        jnp.concatenate([adst2.reshape(N), zpad]),
                        bmax2, hp2)
    return _final(acc2, b2)


# speedup vs baseline: 2.4622x; 1.1153x over previous
import functools

import jax
import jax.numpy as jnp
from jax import lax
from jax.experimental import pallas as pl
from jax.experimental.pallas import tpu as pltpu
from jax.experimental.pallas import tpu_sc as plsc

N = 10000
E = 320000
D = 128
DP = 144
NEG_SLOPE = 0.2

NC = 2
NS = 16
NW = NC * NS
C = 128
NCHUNK = 80
EPT = C * NCHUNK
E_PAD = NW * EPT
NA = N + C
RPT = N // NS


def _dense_block(h, a2_ref, hp_ref, asrc_ref, adst_ref, bmax_ref):
    hp_ref[:, :D] = h
    col = lax.broadcasted_iota(jnp.int32, (N, DP - D), 1)
    hp_ref[:, D:] = jnp.where(col == 0, 1.0, 0.0)
    asd = jnp.dot(h, a2_ref[...], preferred_element_type=jnp.float32)
    asrc_ref[...] = asd[:, 0:1]
    adst_ref[...] = asd[:, 1:2]
    s = jnp.max(asd[:, 0:1]) + jnp.max(asd[:, 1:2])
    b = jnp.where(s > 0.0, s, NEG_SLOPE * s)
    bmax_ref[...] = jnp.full((16,), b, jnp.float32)


def _dense1_body(x_ref, w_ref, a2_ref, hp_ref, asrc_ref, adst_ref, bmax_ref):
    h = jnp.dot(x_ref[...], w_ref[...], preferred_element_type=jnp.float32)
    _dense_block(h, a2_ref, hp_ref, asrc_ref, adst_ref, bmax_ref)


def _mid_body(acc_ref, b1_ref, w_ref, a2_ref,
              hp_ref, asrc_ref, adst_ref, bmax_ref):
    acc = acc_ref[:N] + acc_ref[N:]
    h1 = acc[:, :D] / (acc[:, D:D + 1] + 1e-16) + b1_ref[...]
    h1 = jnp.where(h1 > 0.0, h1, jnp.exp(h1) - 1.0)
    h = jnp.dot(h1, w_ref[...], preferred_element_type=jnp.float32)
    _dense_block(h, a2_ref, hp_ref, asrc_ref, adst_ref, bmax_ref)


def _final_body(acc_ref, b2_ref, out_ref):
    acc = acc_ref[:N] + acc_ref[N:]
    out_ref[...] = acc[:, :D] / (acc[:, D:D + 1] + 1e-16) + b2_ref[...]


_stage_out = [
    jax.ShapeDtypeStruct((N, DP), jnp.float32),
    jax.ShapeDtypeStruct((N, 1), jnp.float32),
    jax.ShapeDtypeStruct((N, 1), jnp.float32),
    jax.ShapeDtypeStruct((16,), jnp.float32),
]

_dense1 = pl.pallas_call(_dense1_body, out_shape=_stage_out)
_mid = pl.pallas_call(_mid_body, out_shape=_stage_out)
_final = pl.pallas_call(
    _final_body, out_shape=jax.ShapeDtypeStruct((N, D), jnp.float32))


_mesh = plsc.VectorSubcoreMesh(core_axis_name="c", subcore_axis_name="s")


@functools.partial(
    pl.kernel,
    out_type=jax.ShapeDtypeStruct((NC * N, DP), jnp.float32),
    mesh=_mesh,
    scratch_types=[
        pltpu.VMEM((16,), jnp.float32),
        [pltpu.VMEM((2 * C,), jnp.int32)] * 2,
        [pltpu.VMEM((C,), jnp.float32)] * 2,
        [pltpu.VMEM((C,), jnp.float32)] * 2,
        [pltpu.VMEM((C,), jnp.float32)] * 2,
        [pltpu.VMEM((C, DP), jnp.float32)] * 2,
        pltpu.VMEM_SHARED((NA, DP), jnp.float32),
        [pltpu.SemaphoreType.DMA] * 2,
        [pltpu.SemaphoreType.DMA] * 2,
    ],
    compiler_params=pltpu.CompilerParams(
        use_tc_tiling_on_sc=False, needs_layout_passes=False),
)
def _edge_kernel(sd_hbm, asrc_hbm, adst_hbm, bmax_hbm, hp_hbm,
                 out_hbm, bmax_v, sd_vs, asg_vs, adg_vs, w_vs,
                 rows_vs, acc_sp, gsems, ssems):
    cid = lax.axis_index("c")
    sid = lax.axis_index("s")
    wid = sid * NC + cid

    pltpu.sync_copy(bmax_hbm, bmax_v)
    bmax = bmax_v[...]
    rows_v = rows_vs[0]

    zero16 = jnp.zeros((16,), jnp.float32)

    def _zbody(r, carry):
        for k in range(DP // 16):
            rows_v[r, pl.ds(k * 16, 16)] = zero16
        return carry

    lax.fori_loop(0, C, _zbody, 0)
    row0 = sid * RPT
    for i in range(RPT // C):
        pltpu.sync_copy(rows_v, acc_sp.at[pl.ds(row0 + i * C, C)])
    rem = RPT % C
    if rem:
        pltpu.sync_copy(rows_v.at[pl.ds(0, rem)],
                        acc_sp.at[pl.ds(row0 + RPT - rem, rem)])
    plsc.subcore_barrier()

    e0 = wid * EPT

    def _issue(ci, b, drain):
        if drain:
            pltpu.make_async_copy(rows_vs[b],
                                  acc_sp.at[sd_vs[b].at[pl.ds(0, C)]],
                                  ssems[b]).wait()
        base = (e0 + ci * C) * 2
        pltpu.sync_copy(sd_hbm.at[pl.ds(base, 2 * C)], sd_vs[b])
        src_i = sd_vs[b].at[pl.ds(C, C)]
        dst_i = sd_vs[b].at[pl.ds(0, C)]
        pltpu.async_copy(hp_hbm.at[src_i], rows_vs[b], gsems[b])
        pltpu.async_copy(asrc_hbm.at[src_i], asg_vs[b], gsems[b])
        pltpu.async_copy(adst_hbm.at[dst_i], adg_vs[b], gsems[b])

    def _consume(b):
        src_i = sd_vs[b].at[pl.ds(C, C)]
        dst_i = sd_vs[b].at[pl.ds(0, C)]
        pltpu.make_async_copy(hp_hbm.at[src_i], rows_vs[b],
                              gsems[b]).wait()
        pltpu.make_async_copy(asrc_hbm.at[src_i], asg_vs[b],
                              gsems[b]).wait()
        pltpu.make_async_copy(adst_hbm.at[dst_i], adg_vs[b],
                              gsems[b]).wait()
        for j in range(C // 16):
            e = (asg_vs[b][pl.ds(j * 16, 16)]
                 + adg_vs[b][pl.ds(j * 16, 16)])
            e = jnp.where(e > 0.0, e, NEG_SLOPE * e)
            w_vs[b][pl.ds(j * 16, 16)] = jnp.exp(e - bmax)

        @plsc.parallel_loop(0, C, step=1, unroll=4)
        def _scale(r):
            wb = plsc.load_gather(w_vs[b], [jnp.full((16,), r, jnp.int32)])
            for k in range(DP // 16):
                rows_vs[b][r, pl.ds(k * 16, 16)] = (
                    rows_vs[b][r, pl.ds(k * 16, 16)] * wb)

        pltpu.async_copy(rows_vs[b], acc_sp.at[sd_vs[b].at[pl.ds(0, C)]],
                         ssems[b], add=True)

    _issue(0, 0, False)
    _issue(1, 1, False)

    def _pair(k, carry):
        ci = 2 * k
        _consume(0)
        _issue(ci + 2, 0, True)
        _consume(1)
        _issue(ci + 3, 1, True)
        return carry

    lax.fori_loop(0, NCHUNK // 2 - 1, _pair, 0)
    _consume(0)
    _consume(1)
    pltpu.make_async_copy(rows_vs[0], acc_sp.at[sd_vs[0].at[pl.ds(0, C)]],
                          ssems[0]).wait()
    pltpu.make_async_copy(rows_vs[1], acc_sp.at[sd_vs[1].at[pl.ds(0, C)]],
                          ssems[1]).wait()
    plsc.subcore_barrier()

    out0 = cid * N + row0
    for i in range(RPT // C):
        pltpu.sync_copy(acc_sp.at[pl.ds(row0 + i * C, C)], rows_v)
        pltpu.sync_copy(rows_v, out_hbm.at[pl.ds(out0 + i * C, C)])
    if rem:
        pltpu.sync_copy(acc_sp.at[pl.ds(row0 + RPT - rem, rem)],
                        rows_v.at[pl.ds(0, rem)])
        pltpu.sync_copy(rows_v.at[pl.ds(0, rem)],
                        out_hbm.at[pl.ds(out0 + RPT - rem, rem)])


def kernel(x, edge_index, W1, a_src1, a_dst1, b1, W2, a_src2, a_dst2, b2):
    pad = E_PAD - E
    ar = jnp.arange(pad, dtype=jnp.int32)
    src = jnp.concatenate([edge_index[0], ar % N])
    dst = jnp.concatenate([edge_index[1], N + (ar % C)])
    sd = jnp.concatenate(
        [dst.reshape(-1, C), src.reshape(-1, C)], axis=1).reshape(-1)
    a21 = jnp.concatenate([a_src1, a_dst1], axis=0).T
    a22 = jnp.concatenate([a_src2, a_dst2], axis=0).T

    zpad = jnp.zeros((NA - N,), jnp.float32)

    hp1, asrc1, adst1, bmax1 = _dense1(x, W1, a21)
    acc1 = _edge_kernel(sd, asrc1.reshape(N),
                        jnp.concatenate([adst1.reshape(N), zpad]),
                        bmax1, hp1)
    hp2, asrc2, adst2, bmax2 = _mid(acc1, b1, W2, a22)
    acc2 = _edge_kernel(sd, asrc2.reshape(N),
                        jnp.concatenate([adst2.reshape(N), zpad]),
                        bmax2, hp2)
    return _final(acc2, b2)
